# baseline (speedup 1.0000x reference)
import functools

import jax
import jax.numpy as jnp
from jax import lax
from jax.experimental import pallas as pl
from jax.experimental.pallas import tpu as pltpu
from jax.experimental.pallas import tpu_sc as plsc

BATCH = 16384
EMB_DIM = 64
PAIR = 2 * EMB_DIM
NUM_CORES = 2
NUM_SUBCORES = 16
NUM_WORKERS = NUM_CORES * NUM_SUBCORES
ROWS_PER_W = BATCH // NUM_WORKERS
CHUNK = 128
N_CHUNKS = ROWS_PER_W // CHUNK
LANES = 16


@functools.cache
def _sc_gather_kernel():
    mesh = plsc.VectorSubcoreMesh(core_axis_name="c", subcore_axis_name="s")

    @functools.partial(
        pl.kernel,
        mesh=mesh,
        out_type=[
            jax.ShapeDtypeStruct((BATCH, PAIR), jnp.float32),
            jax.ShapeDtypeStruct((BATCH, PAIR), jnp.float32),
            jax.ShapeDtypeStruct((BATCH,), jnp.float32),
            jax.ShapeDtypeStruct((BATCH,), jnp.float32),
        ],
        scratch_types=[
            pltpu.VMEM((ROWS_PER_W,), jnp.int32),
            pltpu.VMEM((ROWS_PER_W,), jnp.int32),
            pltpu.VMEM((ROWS_PER_W,), jnp.float32),
            pltpu.VMEM((CHUNK, PAIR), jnp.float32),
            pltpu.SemaphoreType.DMA,
        ],
    )
    def _sc_gather(uids_hbm, iids_hbm, utab_hbm, itab_hbm,
                   uout_hbm, iout_hbm, uflag_hbm, iflag_hbm,
                   ids_v, tid_v, flag_v, tiles_v, sem):
        wid = lax.axis_index("s") * NUM_CORES + lax.axis_index("c")
        base = wid * ROWS_PER_W

        def one_table(ids_hbm, tab_hbm, out_hbm, flag_hbm):
            pltpu.sync_copy(ids_hbm.at[pl.ds(base, ROWS_PER_W)], ids_v)
            for k in range(ROWS_PER_W // LANES):
                ids = ids_v[pl.ds(k * LANES, LANES)]
                tid_v[pl.ds(k * LANES, LANES)] = ids >> 1
                flag_v[pl.ds(k * LANES, LANES)] = (ids & 1).astype(jnp.float32)
            for j in range(N_CHUNKS):
                pltpu.async_copy(
                    tab_hbm.at[tid_v.at[pl.ds(j * CHUNK, CHUNK)]],
                    tiles_v, sem).wait()
                pltpu.sync_copy(
                    tiles_v, out_hbm.at[pl.ds(base + j * CHUNK, CHUNK)])
            pltpu.sync_copy(flag_v, flag_hbm.at[pl.ds(base, ROWS_PER_W)])

        one_table(uids_hbm, utab_hbm, uout_hbm, uflag_hbm)
        one_table(iids_hbm, itab_hbm, iout_hbm, iflag_hbm)

    return _sc_gather


def _tc_body(u_ref, i_ref, uf_ref, if_ref, wu_ref, bu_ref, wi_ref, bi_ref,
             o_ref):
    dn = (((1,), (1,)), ((), ()))
    up = u_ref[...]
    u_row = jnp.where(uf_ref[...] > 0.5, up[:, EMB_DIM:], up[:, :EMB_DIM])
    ip = i_ref[...]
    i_row = jnp.where(if_ref[...] > 0.5, ip[:, EMB_DIM:], ip[:, :EMB_DIM])
    u = lax.dot_general(u_row, wu_ref[...], dn,
                        preferred_element_type=jnp.float32) + bu_ref[...]
    u = jnp.maximum(u, 0.0)
    i = lax.dot_general(i_row, wi_ref[...], dn,
                        preferred_element_type=jnp.float32) + bi_ref[...]
    i = jnp.maximum(i, 0.0)
    un = jnp.sqrt(jnp.sum(u * u, axis=1, keepdims=True))
    inn = jnp.sqrt(jnp.sum(i * i, axis=1, keepdims=True))
    denom = jnp.maximum(un, 1e-12) * jnp.maximum(inn, 1e-12)
    o_ref[...] = jnp.sum(u * i, axis=1, keepdims=True) / denom


_TC_BLOCK = 2048


def _tc_scores(u_rows, i_rows, uf, if_, Wu, bu2, Wi, bi2):
    grid = (BATCH // _TC_BLOCK,)
    return pl.pallas_call(
        _tc_body,
        grid=grid,
        in_specs=[
            pl.BlockSpec((_TC_BLOCK, PAIR), lambda g: (g, 0)),
            pl.BlockSpec((_TC_BLOCK, PAIR), lambda g: (g, 0)),
            pl.BlockSpec((_TC_BLOCK, 1), lambda g: (g, 0)),
            pl.BlockSpec((_TC_BLOCK, 1), lambda g: (g, 0)),
            pl.BlockSpec((EMB_DIM, EMB_DIM), lambda g: (0, 0)),
            pl.BlockSpec((1, EMB_DIM), lambda g: (0, 0)),
            pl.BlockSpec((EMB_DIM, EMB_DIM), lambda g: (0, 0)),
            pl.BlockSpec((1, EMB_DIM), lambda g: (0, 0)),
        ],
        out_specs=pl.BlockSpec((_TC_BLOCK, 1), lambda g: (g, 0)),
        out_shape=jax.ShapeDtypeStruct((BATCH, 1), jnp.float32),
    )(u_rows, i_rows, uf, if_, Wu, bu2, Wi, bi2)


def kernel(user_ids, item_ids, user_emb, item_emb, Wu, bu, Wi, bi):
    uids = user_ids.astype(jnp.int32)
    iids = item_ids.astype(jnp.int32)
    utab = user_emb.reshape(user_emb.shape[0] // 2, PAIR)
    itab = item_emb.reshape(item_emb.shape[0] // 2, PAIR)
    u_rows, i_rows, uf, if_ = _sc_gather_kernel()(uids, iids, utab, itab)
    scores = _tc_scores(u_rows, i_rows, uf.reshape(BATCH, 1),
                        if_.reshape(BATCH, 1), Wu, bu.reshape(1, EMB_DIM),
                        Wi, bi.reshape(1, EMB_DIM))
    return scores.reshape(BATCH)


# speedup vs baseline: 1.6076x; 1.6076x over previous
import functools

import jax
import jax.numpy as jnp
from jax import lax
from jax.experimental import pallas as pl
from jax.experimental.pallas import tpu as pltpu
from jax.experimental.pallas import tpu_sc as plsc

BATCH = 16384
EMB_DIM = 64
PAIR = 2 * EMB_DIM
NUM_CORES = 2
NUM_SUBCORES = 16
NUM_WORKERS = NUM_CORES * NUM_SUBCORES
ROWS_PER_W = BATCH // NUM_WORKERS
CHUNK = 128
N_CHUNKS = ROWS_PER_W // CHUNK
LANES = 16


@functools.cache
def _sc_gather_kernel():
    mesh = plsc.VectorSubcoreMesh(core_axis_name="c", subcore_axis_name="s")

    @functools.partial(
        pl.kernel,
        mesh=mesh,
        out_type=[
            jax.ShapeDtypeStruct((BATCH, PAIR), jnp.float32),
            jax.ShapeDtypeStruct((BATCH, PAIR), jnp.float32),
            jax.ShapeDtypeStruct((BATCH,), jnp.float32),
            jax.ShapeDtypeStruct((BATCH,), jnp.float32),
        ],
        scratch_types=[
            pltpu.VMEM((ROWS_PER_W,), jnp.int32),
            pltpu.VMEM((ROWS_PER_W,), jnp.int32),
            pltpu.VMEM((ROWS_PER_W,), jnp.float32),
            pltpu.VMEM((CHUNK, PAIR), jnp.float32),
            pltpu.SemaphoreType.DMA,
        ],
    )
    def _sc_gather(uids_hbm, iids_hbm, utab_hbm, itab_hbm,
                   uout_hbm, iout_hbm, uflag_hbm, iflag_hbm,
                   ids_v, tid_v, flag_v, tiles_v, sem):
        wid = lax.axis_index("s") * NUM_CORES + lax.axis_index("c")
        base = wid * ROWS_PER_W

        def one_table(ids_hbm, tab_hbm, out_hbm, flag_hbm):
            pltpu.sync_copy(ids_hbm.at[pl.ds(base, ROWS_PER_W)], ids_v)
            for k in range(ROWS_PER_W // LANES):
                ids = ids_v[pl.ds(k * LANES, LANES)]
                tid_v[pl.ds(k * LANES, LANES)] = (
                    ((ids >> 12) << 11) | (ids & 2047))
                flag_v[pl.ds(k * LANES, LANES)] = (
                    (ids >> 11) & 1).astype(jnp.float32)
            for j in range(N_CHUNKS):
                pltpu.async_copy(
                    tab_hbm.at[tid_v.at[pl.ds(j * CHUNK, CHUNK)]],
                    tiles_v, sem).wait()
                pltpu.sync_copy(
                    tiles_v, out_hbm.at[pl.ds(base + j * CHUNK, CHUNK)])
            pltpu.sync_copy(flag_v, flag_hbm.at[pl.ds(base, ROWS_PER_W)])

        one_table(uids_hbm, utab_hbm, uout_hbm, uflag_hbm)
        one_table(iids_hbm, itab_hbm, iout_hbm, iflag_hbm)

    return _sc_gather


_RP_LANES = 4096
_RP_ROWS = _RP_LANES // 2
_RP_GRID = -(-1000000 // _RP_LANES)
TAB_ROWS = _RP_GRID * _RP_ROWS


def _repack_body(t_ref, o_ref):
    b = t_ref[...].T
    o_ref[...] = jnp.concatenate([b[:_RP_ROWS], b[_RP_ROWS:]], axis=1)


def _repack(tabT):
    return pl.pallas_call(
        _repack_body,
        grid=(_RP_GRID,),
        in_specs=[pl.BlockSpec((EMB_DIM, _RP_LANES), lambda g: (0, g))],
        out_specs=pl.BlockSpec((_RP_ROWS, PAIR), lambda g: (g, 0)),
        out_shape=jax.ShapeDtypeStruct((TAB_ROWS, PAIR), jnp.float32),
    )(tabT)


def _tc_body(u_ref, i_ref, uf_ref, if_ref, wu_ref, bu_ref, wi_ref, bi_ref,
             o_ref):
    dn = (((1,), (1,)), ((), ()))
    up = u_ref[...]
    u_row = jnp.where(uf_ref[...] > 0.5, up[:, EMB_DIM:], up[:, :EMB_DIM])
    ip = i_ref[...]
    i_row = jnp.where(if_ref[...] > 0.5, ip[:, EMB_DIM:], ip[:, :EMB_DIM])
    u = lax.dot_general(u_row, wu_ref[...], dn,
                        preferred_element_type=jnp.float32) + bu_ref[...]
    u = jnp.maximum(u, 0.0)
    i = lax.dot_general(i_row, wi_ref[...], dn,
                        preferred_element_type=jnp.float32) + bi_ref[...]
    i = jnp.maximum(i, 0.0)
    un = jnp.sqrt(jnp.sum(u * u, axis=1, keepdims=True))
    inn = jnp.sqrt(jnp.sum(i * i, axis=1, keepdims=True))
    denom = jnp.maximum(un, 1e-12) * jnp.maximum(inn, 1e-12)
    o_ref[...] = jnp.sum(u * i, axis=1, keepdims=True) / denom


_TC_BLOCK = 2048


def _tc_scores(u_rows, i_rows, uf, if_, Wu, bu2, Wi, bi2):
    grid = (BATCH // _TC_BLOCK,)
    return pl.pallas_call(
        _tc_body,
        grid=grid,
        in_specs=[
            pl.BlockSpec((_TC_BLOCK, PAIR), lambda g: (g, 0)),
            pl.BlockSpec((_TC_BLOCK, PAIR), lambda g: (g, 0)),
            pl.BlockSpec((_TC_BLOCK, 1), lambda g: (g, 0)),
            pl.BlockSpec((_TC_BLOCK, 1), lambda g: (g, 0)),
            pl.BlockSpec((EMB_DIM, EMB_DIM), lambda g: (0, 0)),
            pl.BlockSpec((1, EMB_DIM), lambda g: (0, 0)),
            pl.BlockSpec((EMB_DIM, EMB_DIM), lambda g: (0, 0)),
            pl.BlockSpec((1, EMB_DIM), lambda g: (0, 0)),
        ],
        out_specs=pl.BlockSpec((_TC_BLOCK, 1), lambda g: (g, 0)),
        out_shape=jax.ShapeDtypeStruct((BATCH, 1), jnp.float32),
    )(u_rows, i_rows, uf, if_, Wu, bu2, Wi, bi2)


def kernel(user_ids, item_ids, user_emb, item_emb, Wu, bu, Wi, bi):
    uids = user_ids.astype(jnp.int32)
    iids = item_ids.astype(jnp.int32)
    utab = _repack(user_emb.T)
    itab = _repack(item_emb.T)
    u_rows, i_rows, uf, if_ = _sc_gather_kernel()(uids, iids, utab, itab)
    scores = _tc_scores(u_rows, i_rows, uf.reshape(BATCH, 1),
                        if_.reshape(BATCH, 1), Wu, bu.reshape(1, EMB_DIM),
                        Wi, bi.reshape(1, EMB_DIM))
    return scores.reshape(BATCH)
